# baseline (speedup 1.0000x reference)
import jax
import jax.numpy as jnp
from jax.experimental import pallas as pl

def _dot_bf16(a, b):
    return jnp.dot(a, b, preferred_element_type=jnp.float32)


def _mm_a_kernel(x_ref, w1_ref, o_ref):
    o_ref[...] = _dot_bf16(x_ref[...], w1_ref[...])


def _gc1_kernel(adj_ref, s1_ref, b1_ref, w2_ref, o_ref):
    acc = _dot_bf16(adj_ref[...], s1_ref[...])
    h = jnp.maximum(acc + b1_ref[...], 0.0)
    o_ref[...] = _dot_bf16(h, w2_ref[...])


def _gc2_kernel(adj_ref, s2_ref, b2_ref, o_ref):
    acc = _dot_bf16(adj_ref[...], s2_ref[...])
    acc = acc + b2_ref[...]
    m = jnp.max(acc, axis=1, keepdims=True)
    e = jnp.exp(acc - m)
    o_ref[...] = e / jnp.sum(e, axis=1, keepdims=True)


def _gcn_forward(x, adj, W1, b1, W2, b2, bm_a, bm, interpret=False, stop_after=None):
    n, f_in = x.shape
    hid = W1.shape[1]
    c = W2.shape[1]
    b1r = b1.reshape(1, hid)
    b2r = b2.reshape(1, c)

    s1 = pl.pallas_call(
        _mm_a_kernel,
        grid=(n // bm_a,),
        in_specs=[
            pl.BlockSpec((bm_a, f_in), lambda i: (i, 0)),
            pl.BlockSpec((f_in, hid), lambda i: (0, 0)),
        ],
        out_specs=pl.BlockSpec((bm_a, hid), lambda i: (i, 0)),
        out_shape=jax.ShapeDtypeStruct((n, hid), jnp.float32),
        interpret=interpret,
    )(x, W1)

    if stop_after == "A":
        return s1
    s2 = pl.pallas_call(
        _gc1_kernel,
        grid=(n // bm,),
        in_specs=[
            pl.BlockSpec((bm, n), lambda i: (i, 0)),
            pl.BlockSpec((n, hid), lambda i: (0, 0)),
            pl.BlockSpec((1, hid), lambda i: (0, 0)),
            pl.BlockSpec((hid, c), lambda i: (0, 0)),
        ],
        out_specs=pl.BlockSpec((bm, c), lambda i: (i, 0)),
        out_shape=jax.ShapeDtypeStruct((n, c), jnp.float32),
        interpret=interpret,
    )(adj, s1, b1r, W2)
    if stop_after == "B":
        return s2

    out = pl.pallas_call(
        _gc2_kernel,
        grid=(n // bm,),
        in_specs=[
            pl.BlockSpec((bm, n), lambda i: (i, 0)),
            pl.BlockSpec((n, c), lambda i: (0, 0)),
            pl.BlockSpec((1, c), lambda i: (0, 0)),
        ],
        out_specs=pl.BlockSpec((bm, c), lambda i: (i, 0)),
        out_shape=jax.ShapeDtypeStruct((n, c), jnp.float32),
        interpret=interpret,
    )(adj, s2, b2r)
    return out


def kernel(x, adj, W1, b1, W2, b2):
    return _gcn_forward(x, adj, W1, b1, W2, b2, bm_a=2000, bm=400,
                        stop_after=None)


# speedup vs baseline: 1.0239x; 1.0239x over previous
import jax
import jax.numpy as jnp
from jax.experimental import pallas as pl


def _mm_a_kernel(x_ref, w1_ref, o_ref):
    xb = x_ref[...].astype(jnp.bfloat16)
    o_ref[...] = jnp.dot(xb, w1_ref[...],
                         preferred_element_type=jnp.float32).astype(jnp.bfloat16)


def _gc1_kernel(adj_ref, s1_ref, b1_ref, w2_ref, o_ref):
    ab = adj_ref[...].astype(jnp.bfloat16)
    acc = jnp.dot(ab, s1_ref[...], preferred_element_type=jnp.float32)
    h = jnp.maximum(acc + b1_ref[...], 0.0).astype(jnp.bfloat16)
    o_ref[...] = jnp.dot(h, w2_ref[...],
                         preferred_element_type=jnp.float32).astype(jnp.bfloat16)


def _gc2_kernel(adj_ref, s2_ref, b2_ref, o_ref):
    ab = adj_ref[...].astype(jnp.bfloat16)
    acc = jnp.dot(ab, s2_ref[...], preferred_element_type=jnp.float32)
    acc = acc + b2_ref[...]
    m = jnp.max(acc, axis=1, keepdims=True)
    e = jnp.exp(acc - m)
    o_ref[...] = e / jnp.sum(e, axis=1, keepdims=True)


def _gcn_forward(x, adj, W1, b1, W2, b2, bm_a, bm, interpret=False):
    n, f_in = x.shape
    hid = W1.shape[1]
    c = W2.shape[1]
    b1r = b1.reshape(1, hid)
    b2r = b2.reshape(1, c)
    W1b = W1.astype(jnp.bfloat16)
    W2b = W2.astype(jnp.bfloat16)

    s1 = pl.pallas_call(
        _mm_a_kernel,
        grid=(n // bm_a,),
        in_specs=[
            pl.BlockSpec((bm_a, f_in), lambda i: (i, 0)),
            pl.BlockSpec((f_in, hid), lambda i: (0, 0)),
        ],
        out_specs=pl.BlockSpec((bm_a, hid), lambda i: (i, 0)),
        out_shape=jax.ShapeDtypeStruct((n, hid), jnp.bfloat16),
        interpret=interpret,
    )(x, W1b)

    s2 = pl.pallas_call(
        _gc1_kernel,
        grid=(n // bm,),
        in_specs=[
            pl.BlockSpec((bm, n), lambda i: (i, 0)),
            pl.BlockSpec((n, hid), lambda i: (0, 0)),
            pl.BlockSpec((1, hid), lambda i: (0, 0)),
            pl.BlockSpec((hid, c), lambda i: (0, 0)),
        ],
        out_specs=pl.BlockSpec((bm, c), lambda i: (i, 0)),
        out_shape=jax.ShapeDtypeStruct((n, c), jnp.bfloat16),
        interpret=interpret,
    )(adj, s1, b1r, W2b)

    out = pl.pallas_call(
        _gc2_kernel,
        grid=(n // bm,),
        in_specs=[
            pl.BlockSpec((bm, n), lambda i: (i, 0)),
            pl.BlockSpec((n, c), lambda i: (0, 0)),
            pl.BlockSpec((1, c), lambda i: (0, 0)),
        ],
        out_specs=pl.BlockSpec((bm, c), lambda i: (i, 0)),
        out_shape=jax.ShapeDtypeStruct((n, c), jnp.float32),
        interpret=interpret,
    )(adj, s2, b2r)
    return out


def kernel(x, adj, W1, b1, W2, b2):
    return _gcn_forward(x, adj, W1, b1, W2, b2, bm_a=2000, bm=400)
